# baseline (speedup 1.0000x reference)
import functools

import jax
import jax.numpy as jnp
from jax import lax
from jax.experimental import pallas as pl
from jax.experimental.pallas import tpu as pltpu
from jax.experimental.pallas import tpu_sc as plsc

NC, NS, LANES = 2, 16, 16
NW = NC * NS
D = 64
CHUNK = 128


def _emb_body(T, n_chunks, idx_hbm, tok_hbm, pos_hbm, out_hbm,
              idx_v, tok_a, tok_b, pos_v, g_a, g_b, s_a, s_b):
    wid = lax.axis_index("s") * NC + lax.axis_index("c")
    rows_per_w = n_chunks * CHUNK
    chunks_per_t = T // CHUNK

    pltpu.sync_copy(idx_hbm.at[wid], idx_v)

    tok_bufs = (tok_a, tok_b)
    g_sems = (g_a, g_b)
    s_sems = (s_a, s_b)

    def chunk_of(u):
        return (u // 2) + (u % 2) * chunks_per_t

    def start_gather(u):
        b = u % 2
        pltpu.async_copy(tok_hbm.at[idx_v.at[chunk_of(u)]], tok_bufs[b], g_sems[b])

    def wait_gather(u):
        b = u % 2
        pltpu.make_async_copy(tok_hbm.at[idx_v.at[chunk_of(u)]], tok_bufs[b],
                              g_sems[b]).wait()

    def out_slice(u):
        row0 = wid * rows_per_w + chunk_of(u) * CHUNK
        return out_hbm.at[pl.ds(row0, CHUNK), :]

    def start_store(u):
        b = u % 2
        pltpu.async_copy(tok_bufs[b], out_slice(u), s_sems[b])

    def wait_store(u):
        b = u % 2
        pltpu.make_async_copy(tok_bufs[b], out_slice(u), s_sems[b]).wait()

    start_gather(0)
    n_units = n_chunks
    for u in range(n_units):
        buf = tok_bufs[u % 2]
        wait_gather(u)
        if u % 2 == 0:
            t0 = (u // 2) * CHUNK
            pltpu.sync_copy(pos_hbm.at[pl.ds(t0, CHUNK), :], pos_v)

        def add_row(i, _):
            for j in range(D // LANES):
                sl = pl.ds(j * LANES, LANES)
                buf[i, sl] = buf[i, sl] + pos_v[i, sl]
            return 0

        lax.fori_loop(0, CHUNK, add_row, 0)

        if u >= 1:
            wait_store(u - 1)
        if u + 1 < n_units:
            start_gather(u + 1)
        start_store(u)
    wait_store(n_units - 1)


def kernel(idx, token_table, position_table):
    B, T = idx.shape
    V, d = token_table.shape
    total = B * T
    assert d == D and total % (NW * CHUNK) == 0 and T % CHUNK == 0
    n_chunks = total // (NW * CHUNK)

    idx3 = idx.reshape(NW, n_chunks, CHUNK).astype(jnp.int32)

    mesh = plsc.VectorSubcoreMesh(core_axis_name="c", subcore_axis_name="s")
    body = functools.partial(_emb_body, T, n_chunks)
    out = pl.kernel(
        body,
        out_type=jax.ShapeDtypeStruct((total, D), jnp.float32),
        mesh=mesh,
        compiler_params=pltpu.CompilerParams(use_tc_tiling_on_sc=False),
        scratch_types=[
            pltpu.VMEM((n_chunks, CHUNK), jnp.int32),
            pltpu.VMEM((CHUNK, D), jnp.float32),
            pltpu.VMEM((CHUNK, D), jnp.float32),
            pltpu.VMEM((CHUNK, D), jnp.float32),
            pltpu.SemaphoreType.DMA,
            pltpu.SemaphoreType.DMA,
            pltpu.SemaphoreType.DMA,
            pltpu.SemaphoreType.DMA,
        ],
    )(idx3, token_table, position_table)
    return out.reshape(B, T, D)
